# Initial kernel scaffold; baseline (speedup 1.0000x reference)
import jax
import jax.numpy as jnp
from jax.experimental import pallas as pl


def kernel(psi, u):
    raise NotImplementedError("write your pallas kernel here")


# speedup vs baseline: 39.3880x; 39.3880x over previous
import functools

import jax
import jax.numpy as jnp
from jax import lax
from jax.experimental import pallas as pl
from jax.experimental.pallas import tpu as pltpu
from jax.experimental.pallas import tpu_sc as plsc

N = 1 << 22
ROWS = 2048
COLS = 2048
HALF = 1024
NC, NS = 2, 16
L = 16

P1_ROWS_PER_TILE = ROWS // NS
P1_CHUNK = 16
P2_ROWS_PER_TILE = ROWS // (NC * NS)
P2_CHUNK = 16


def _sq_accum_row(buf, r, base, accs):
    def body(i, a):
        a0, a1, a2, a3 = a
        x0 = buf[r, pl.ds(base + i * 64, L)]
        x1 = buf[r, pl.ds(base + i * 64 + 16, L)]
        x2 = buf[r, pl.ds(base + i * 64 + 32, L)]
        x3 = buf[r, pl.ds(base + i * 64 + 48, L)]
        return (a0 + x0 * x0, a1 + x1 * x1, a2 + x2 * x2, a3 + x3 * x3)
    return lax.fori_loop(0, HALF // 64, body, accs, unroll=4)


_mesh = plsc.VectorSubcoreMesh(core_axis_name="c", subcore_axis_name="s",
                               num_cores=NC, num_subcores=NS)


@functools.partial(
    pl.kernel,
    out_type=(
        jax.ShapeDtypeStruct((ROWS, HALF), jnp.float32),
        jax.ShapeDtypeStruct((2, L), jnp.float32),
    ),
    mesh=_mesh,
    scratch_types=[
        pltpu.VMEM((P1_CHUNK, COLS), jnp.float32),
        pltpu.VMEM((P2_CHUNK, HALF), jnp.float32),
        pltpu.VMEM((2, L), jnp.float32),
        pltpu.VMEM((NS, 2, L), jnp.float32),
        pltpu.VMEM((L,), jnp.float32),
        pltpu.VMEM((2, L), jnp.float32),
        pltpu.VMEM_SHARED((NS, 2, L), jnp.float32),
    ],
)
def _sc_measure(psi_hbm, u_hbm, out_hbm, stats_hbm,
                buf1, buf2, part_v, pall_v, u_v, stats_v, shared):
    cid = lax.axis_index("c")
    sid = lax.axis_index("s")
    zero = jnp.zeros((L,), jnp.float32)

    row0 = sid * P1_ROWS_PER_TILE

    def chunk_body(c, accs):
        pltpu.sync_copy(psi_hbm.at[pl.ds(row0 + c * P1_CHUNK, P1_CHUNK), :],
                        buf1)

        def row_body(r, accs):
            a = _sq_accum_row(buf1, r, 0, accs[:4])
            b = _sq_accum_row(buf1, r, HALF, accs[4:])
            return a + b

        return lax.fori_loop(0, P1_CHUNK, row_body, accs)

    accs = lax.fori_loop(0, P1_ROWS_PER_TILE // P1_CHUNK, chunk_body,
                         (zero,) * 8)
    a0 = (accs[0] + accs[1]) + (accs[2] + accs[3])
    a1 = (accs[4] + accs[5]) + (accs[6] + accs[7])

    part_v[0] = a0
    part_v[1] = a1
    pltpu.sync_copy(part_v, shared.at[sid])
    plsc.subcore_barrier()
    pltpu.sync_copy(shared, pall_v)

    def red_body(i, accs):
        a0, a1 = accs
        return (a0 + pall_v[i, 0], a1 + pall_v[i, 1])

    a0, a1 = lax.fori_loop(0, NS, red_body, (zero, zero))
    idx = lax.iota(jnp.int32, L)
    for w in (8, 4, 2, 1):
        a0 = a0 + a0.at[idx ^ w].get(mode="promise_in_bounds")
        a1 = a1 + a1.at[idx ^ w].get(mode="promise_in_bounds")
    s0 = a0[0]
    s1 = a1[0]

    pltpu.sync_copy(u_hbm, u_v)
    u_s = u_v[...][0]

    total = s0 + s1
    outcome = u_s * total > s0
    selected = jnp.where(outcome, s1, s0)
    tb = lax.bitcast_convert_type(total, jnp.int32)
    inv_t = lax.bitcast_convert_type(jnp.int32(0x7EF127EA) - tb, jnp.float32)
    for _ in range(4):
        inv_t = inv_t * (2.0 - total * inv_t)
    p_out = selected * inv_t
    bits = lax.bitcast_convert_type(p_out, jnp.int32)
    y = lax.bitcast_convert_type(jnp.int32(0x5F3759DF) - (bits >> 1),
                                 jnp.float32)
    for _ in range(4):
        y = y * (1.5 - 0.5 * p_out * y * y)
    scale = y

    @pl.when(jnp.logical_and(cid == 0, sid == 0))
    def _():
        stats_v[0] = jnp.full((L,), jnp.where(outcome, 1.0, 0.0), jnp.float32)
        stats_v[1] = jnp.full((L,), p_out, jnp.float32)
        pltpu.sync_copy(stats_v, stats_hbm)

    off = jnp.where(outcome, HALF, 0)
    r2 = (cid * NS + sid) * P2_ROWS_PER_TILE
    for c in range(P2_ROWS_PER_TILE // P2_CHUNK):
        pltpu.sync_copy(
            psi_hbm.at[pl.ds(r2 + c * P2_CHUNK, P2_CHUNK), pl.ds(off, HALF)],
            buf2)

        def srow(r, carry):
            def sbody(i, carry):
                for k in range(4):
                    sl = pl.ds(i * 64 + k * 16, L)
                    buf2[r, sl] = buf2[r, sl] * scale
                return carry
            return lax.fori_loop(0, HALF // 64, sbody, carry, unroll=4)

        lax.fori_loop(0, P2_CHUNK, srow, 0)
        pltpu.sync_copy(buf2,
                        out_hbm.at[pl.ds(r2 + c * P2_CHUNK, P2_CHUNK), :])


def kernel(psi, u):
    psi2d = psi.reshape(ROWS, COLS)
    u16 = jnp.full((L,), u, jnp.float32)
    out2d, stats = _sc_measure(psi2d, u16)
    psi_post = out2d.reshape(N // 2)
    outcome = stats[0, 0] > 0.5
    p_outcome = stats[1, 0]
    return psi_post, outcome, p_outcome


# speedup vs baseline: 45.1219x; 1.1456x over previous
import functools

import jax
import jax.numpy as jnp
from jax import lax
from jax.experimental import pallas as pl
from jax.experimental.pallas import tpu as pltpu
from jax.experimental.pallas import tpu_sc as plsc

N = 1 << 22
ROWS = 2048
COLS = 2048
HALF = 1024
NC, NS = 2, 16
L = 16

P1_ROWS_PER_TILE = ROWS // NS
P1_CHUNK = 16
P2_ROWS_PER_TILE = ROWS // (NC * NS)
P2_CHUNK = 16


def _sq_accum_row(buf, r, base, accs):
    def body(i, a):
        a0, a1, a2, a3 = a
        x0 = buf[r, pl.ds(base + i * 64, L)]
        x1 = buf[r, pl.ds(base + i * 64 + 16, L)]
        x2 = buf[r, pl.ds(base + i * 64 + 32, L)]
        x3 = buf[r, pl.ds(base + i * 64 + 48, L)]
        return (a0 + x0 * x0, a1 + x1 * x1, a2 + x2 * x2, a3 + x3 * x3)
    return lax.fori_loop(0, HALF // 64, body, accs, unroll=4)


_mesh = plsc.VectorSubcoreMesh(core_axis_name="c", subcore_axis_name="s",
                               num_cores=NC, num_subcores=NS)


@functools.partial(
    pl.kernel,
    out_type=(
        jax.ShapeDtypeStruct((ROWS, HALF), jnp.float32),
        jax.ShapeDtypeStruct((2, L), jnp.float32),
    ),
    mesh=_mesh,
    scratch_types=[
        pltpu.VMEM((P1_CHUNK, COLS), jnp.float32),
        pltpu.VMEM((P1_CHUNK, COLS), jnp.float32),
        pltpu.VMEM((P2_CHUNK, HALF), jnp.float32),
        pltpu.VMEM((P2_CHUNK, HALF), jnp.float32),
        pltpu.VMEM((2, L), jnp.float32),
        pltpu.VMEM((NS, 2, L), jnp.float32),
        pltpu.VMEM((L,), jnp.float32),
        pltpu.VMEM((2, L), jnp.float32),
        pltpu.VMEM_SHARED((NS, 2, L), jnp.float32),
        pltpu.SemaphoreType.DMA,
        pltpu.SemaphoreType.DMA,
        pltpu.SemaphoreType.DMA,
        pltpu.SemaphoreType.DMA,
        pltpu.SemaphoreType.DMA,
        pltpu.SemaphoreType.DMA,
    ],
)
def _sc_measure(psi_hbm, u_hbm, out_hbm, stats_hbm,
                buf1a, buf1b, buf2a, buf2b, part_v, pall_v, u_v, stats_v,
                shared, sem1a, sem1b, sem2a, sem2b, semoa, semob):
    cid = lax.axis_index("c")
    sid = lax.axis_index("s")
    zero = jnp.zeros((L,), jnp.float32)

    row0 = sid * P1_ROWS_PER_TILE
    bufs1 = (buf1a, buf1b)
    sems1 = (sem1a, sem1b)
    n1 = P1_ROWS_PER_TILE // P1_CHUNK

    def start1(c):
        b = c % 2
        return pltpu.async_copy(
            psi_hbm.at[pl.ds(row0 + c * P1_CHUNK, P1_CHUNK), :],
            bufs1[b], sems1[b])

    accs = (zero,) * 8
    copies = [start1(0), None]
    for c in range(n1):
        b = c % 2
        copies[b].wait()
        if c + 1 < n1:
            copies[(c + 1) % 2] = start1(c + 1)
        buf = bufs1[b]

        def row_body(r, accs, buf=buf):
            a = _sq_accum_row(buf, r, 0, accs[:4])
            bb = _sq_accum_row(buf, r, HALF, accs[4:])
            return a + bb

        accs = lax.fori_loop(0, P1_CHUNK, row_body, accs)
    a0 = (accs[0] + accs[1]) + (accs[2] + accs[3])
    a1 = (accs[4] + accs[5]) + (accs[6] + accs[7])

    part_v[0] = a0
    part_v[1] = a1
    pltpu.sync_copy(part_v, shared.at[sid])
    plsc.subcore_barrier()
    pltpu.sync_copy(shared, pall_v)

    def red_body(i, accs):
        a0, a1 = accs
        return (a0 + pall_v[i, 0], a1 + pall_v[i, 1])

    a0, a1 = lax.fori_loop(0, NS, red_body, (zero, zero))
    idx = lax.iota(jnp.int32, L)
    for w in (8, 4, 2, 1):
        a0 = a0 + a0.at[idx ^ w].get(mode="promise_in_bounds")
        a1 = a1 + a1.at[idx ^ w].get(mode="promise_in_bounds")
    s0 = a0[0]
    s1 = a1[0]

    pltpu.sync_copy(u_hbm, u_v)
    u_s = u_v[...][0]

    total = s0 + s1
    outcome = u_s * total > s0
    selected = jnp.where(outcome, s1, s0)
    tb = lax.bitcast_convert_type(total, jnp.int32)
    inv_t = lax.bitcast_convert_type(jnp.int32(0x7EF127EA) - tb, jnp.float32)
    for _ in range(4):
        inv_t = inv_t * (2.0 - total * inv_t)
    p_out = selected * inv_t
    bits = lax.bitcast_convert_type(p_out, jnp.int32)
    y = lax.bitcast_convert_type(jnp.int32(0x5F3759DF) - (bits >> 1),
                                 jnp.float32)
    for _ in range(4):
        y = y * (1.5 - 0.5 * p_out * y * y)
    scale = y

    @pl.when(jnp.logical_and(cid == 0, sid == 0))
    def _():
        stats_v[0] = jnp.full((L,), jnp.where(outcome, 1.0, 0.0), jnp.float32)
        stats_v[1] = jnp.full((L,), p_out, jnp.float32)
        pltpu.sync_copy(stats_v, stats_hbm)

    off = jnp.where(outcome, HALF, 0)
    r2 = (cid * NS + sid) * P2_ROWS_PER_TILE
    bufs2 = (buf2a, buf2b)
    sems2 = (sem2a, sem2b)
    semso = (semoa, semob)
    n2 = P2_ROWS_PER_TILE // P2_CHUNK

    def start2(c):
        b = c % 2
        return pltpu.async_copy(
            psi_hbm.at[pl.ds(r2 + c * P2_CHUNK, P2_CHUNK), pl.ds(off, HALF)],
            bufs2[b], sems2[b])

    in_copies = [start2(0), start2(1)]
    out_copies = [None, None]
    for c in range(n2):
        b = c % 2
        in_copies[b].wait()
        buf = bufs2[b]

        def srow(r, carry, buf=buf):
            def sbody(i, carry):
                for k in range(4):
                    sl = pl.ds(i * 64 + k * 16, L)
                    buf[r, sl] = buf[r, sl] * scale
                return carry
            return lax.fori_loop(0, HALF // 64, sbody, carry, unroll=4)

        lax.fori_loop(0, P2_CHUNK, srow, 0)
        out_copies[b] = pltpu.async_copy(
            buf, out_hbm.at[pl.ds(r2 + c * P2_CHUNK, P2_CHUNK), :], semso[b])
        if c + 2 < n2:
            out_copies[b].wait()
            in_copies[b] = start2(c + 2)
    out_copies[0].wait()
    out_copies[1].wait()


def kernel(psi, u):
    psi2d = psi.reshape(ROWS, COLS)
    u16 = jnp.full((L,), u, jnp.float32)
    out2d, stats = _sc_measure(psi2d, u16)
    psi_post = out2d.reshape(N // 2)
    outcome = stats[0, 0] > 0.5
    p_outcome = stats[1, 0]
    return psi_post, outcome, p_outcome
